# baseline (speedup 1.0000x reference)
import jax
import jax.numpy as jnp
from jax import lax
from jax.experimental import pallas as pl
from jax.experimental.pallas import tpu as pltpu
from jax.experimental.pallas import tpu_sc as plsc

BATCH = 4096
LATENT = 32
NROWS = 1000000

_NC, _NS = 2, 16
_NW = _NC * _NS
_BPW = BATCH // _NW


def _gather_body(uids_hbm, iids_hbm, uw_hbm, ub_hbm, iw_hbm, ib_hbm,
                 uw_out, ub_out, iw_out, ib_out,
                 uidx_v, iidx_v, uwv, iwv, ubv, ibv, sem):
    wid = lax.axis_index("s") * _NC + lax.axis_index("c")
    base = wid * _BPW
    pltpu.sync_copy(uids_hbm.at[pl.ds(base, _BPW)], uidx_v)
    pltpu.sync_copy(iids_hbm.at[pl.ds(base, _BPW)], iidx_v)
    copies = [
        pltpu.async_copy(uw_hbm.at[uidx_v], uwv, sem),
        pltpu.async_copy(iw_hbm.at[iidx_v], iwv, sem),
        pltpu.async_copy(ub_hbm.at[uidx_v], ubv, sem),
        pltpu.async_copy(ib_hbm.at[iidx_v], ibv, sem),
    ]
    for c in copies:
        c.wait()
    pltpu.sync_copy(uwv, uw_out.at[pl.ds(base, _BPW)])
    pltpu.sync_copy(iwv, iw_out.at[pl.ds(base, _BPW)])
    pltpu.sync_copy(ubv, ub_out.at[pl.ds(base, _BPW)])
    pltpu.sync_copy(ibv, ib_out.at[pl.ds(base, _BPW)])


_gather_call = pl.kernel(
    _gather_body,
    out_type=(
        jax.ShapeDtypeStruct((BATCH, LATENT), jnp.float32),
        jax.ShapeDtypeStruct((BATCH,), jnp.float32),
        jax.ShapeDtypeStruct((BATCH, LATENT), jnp.float32),
        jax.ShapeDtypeStruct((BATCH,), jnp.float32),
    ),
    mesh=plsc.VectorSubcoreMesh(core_axis_name="c", subcore_axis_name="s"),
    scratch_types=[
        pltpu.VMEM((_BPW,), jnp.int32),
        pltpu.VMEM((_BPW,), jnp.int32),
        pltpu.VMEM((_BPW, LATENT), jnp.float32),
        pltpu.VMEM((_BPW, LATENT), jnp.float32),
        pltpu.VMEM((_BPW,), jnp.float32),
        pltpu.VMEM((_BPW,), jnp.float32),
        pltpu.SemaphoreType.DMA,
    ],
    compiler_params=pltpu.CompilerParams(use_tc_tiling_on_sc=False),
)


def _score_body(uw_ref, iw_ref, ub_ref, ib_ref, bias_ref, out_ref):
    acc = lax.dot_general(uw_ref[...], iw_ref[...],
                          (((1,), (1,)), ((), ())),
                          preferred_element_type=jnp.float32)
    out_ref[...] = acc + ub_ref[...] + ib_ref[...] + bias_ref[...]


_BM = 512


def kernel(user_ids, item_ids, user_weight, user_bias, item_weight,
           item_bias, bias):
    uw, ub, iw, ib = _gather_call(user_ids, item_ids,
                                  user_weight,
                                  jnp.reshape(user_bias, (NROWS,)),
                                  item_weight,
                                  jnp.reshape(item_bias, (NROWS,)))
    ub_col = jnp.reshape(ub, (BATCH, 1))
    ib_row = jnp.reshape(ib, (1, BATCH))
    bias2d = jnp.reshape(bias, (1, 1))
    rating = pl.pallas_call(
        _score_body,
        grid=(BATCH // _BM,),
        in_specs=[
            pl.BlockSpec((_BM, LATENT), lambda i: (i, 0)),
            pl.BlockSpec((BATCH, LATENT), lambda i: (0, 0)),
            pl.BlockSpec((_BM, 1), lambda i: (i, 0)),
            pl.BlockSpec((1, BATCH), lambda i: (0, 0)),
            pl.BlockSpec((1, 1), lambda i: (0, 0)),
        ],
        out_specs=pl.BlockSpec((_BM, BATCH), lambda i: (i, 0)),
        out_shape=jax.ShapeDtypeStruct((BATCH, BATCH), jnp.float32),
    )(uw, iw, ub_col, ib_row, bias2d)
    return rating
